# baseline (speedup 1.0000x reference)
import functools

import jax
import jax.numpy as jnp
from jax import lax
from jax.experimental import pallas as pl
from jax.experimental.pallas import tpu as pltpu
from jax.experimental.pallas import tpu_sc as plsc

_EMB = 300
_LANES = 16
_TILE = 128
_NCHUNK = _EMB // _LANES
_TAIL_OFF = _EMB - _LANES
_BATCH = 25
_SEQ = 200
_IW_OFF = 128
_IW_LEN = _SEQ - _IW_OFF


def _sc_body(tt_hbm, iw_hbm, w_hbm, b_hbm, out_hbm,
             iw_v, blk_v, w_v, b_v, out_v, sem, wsem):
    ns = plsc.get_sparse_core_info().num_subcores
    cid = lax.axis_index("c")
    sid = lax.axis_index("s")
    wid = cid * ns + sid

    @pl.when(wid < _BATCH)
    def _():
        pltpu.sync_copy(iw_hbm.at[wid], iw_v)
        row = iw_v[pl.ds(_SEQ - _LANES, _LANES)][_LANES - 1]
        base = pl.multiple_of((row // _TILE) * _TILE, _TILE)
        off = row - base
        blk_cp = pltpu.async_copy(tt_hbm.at[:, pl.ds(base, _TILE)], blk_v, sem)
        pltpu.async_copy(w_hbm.at[0], w_v, wsem).wait()
        pltpu.sync_copy(b_hbm, b_v.at[pl.ds(0, 1)])
        blk_cp.wait()
        col = jnp.full((_LANES,), off, jnp.int32)
        lanes = lax.iota(jnp.int32, _LANES)
        acc = jnp.zeros((_LANES,), jnp.float32)
        for j in range(_NCHUNK):
            vals = plsc.load_gather(blk_v, [lanes + (j * _LANES), col])
            acc = acc + vals * w_v[pl.ds(j * _LANES, _LANES)]
        tail = plsc.load_gather(blk_v, [lanes + _TAIL_OFF, col])
        tail_w = jnp.where(lanes >= _NCHUNK * _LANES - _TAIL_OFF,
                           w_v[pl.ds(_TAIL_OFF, _LANES)],
                           jnp.zeros((_LANES,), jnp.float32))
        acc = acc + tail * tail_w
        dnums = lax.GatherDimensionNumbers(
            offset_dims=(), collapsed_slice_dims=(0,), start_index_map=(0,))
        for sh in (8, 4, 2, 1):
            perm = lanes ^ sh
            acc = acc + lax.gather(
                acc, perm[:, None], dnums, slice_sizes=(1,),
                mode=lax.GatherScatterMode.PROMISE_IN_BOUNDS)
        bias = b_v[pl.ds(0, _LANES)][0]
        out_v[...] = 1.0 / (1.0 + jnp.exp(-(acc + bias)))
        pltpu.sync_copy(out_v, out_hbm.at[wid])


def kernel(input_words, table, fc_w, fc_b):
    mesh = plsc.VectorSubcoreMesh(core_axis_name="c", subcore_axis_name="s")
    sc_fn = functools.partial(
        pl.kernel,
        mesh=mesh,
        compiler_params=pltpu.CompilerParams(needs_layout_passes=False),
        out_type=jax.ShapeDtypeStruct((_BATCH, _LANES), jnp.float32),
        scratch_types=[
            pltpu.VMEM((_SEQ,), jnp.int32),
            pltpu.VMEM((_EMB, _TILE), jnp.float32),
            pltpu.VMEM((_EMB,), jnp.float32),
            pltpu.VMEM((_LANES,), jnp.float32),
            pltpu.VMEM((_LANES,), jnp.float32),
            pltpu.SemaphoreType.DMA,
            pltpu.SemaphoreType.DMA,
        ],
    )(_sc_body)
    out2d = sc_fn(table.T, input_words.astype(jnp.int32),
                  fc_w.astype(jnp.float32), fc_b.astype(jnp.float32))
    return out2d[:, 0]


# speedup vs baseline: 1.0006x; 1.0006x over previous
import functools

import jax
import jax.numpy as jnp
from jax import lax
from jax.experimental import pallas as pl
from jax.experimental.pallas import tpu as pltpu
from jax.experimental.pallas import tpu_sc as plsc

_EMB = 300
_LANES = 16
_TILE = 128
_NCHUNK = _EMB // _LANES
_TAIL_OFF = _EMB - _LANES
_HALF = 152
_BATCH = 25
_SEQ = 200
_IW_OFF = 128
_IW_LEN = _SEQ - _IW_OFF


def _sc_body(tt_hbm, iw_hbm, w_hbm, b_hbm, out_hbm,
             iw_v, blk_v, w_v, b_v, out_v, sem, sem2, wsem):
    ns = plsc.get_sparse_core_info().num_subcores
    cid = lax.axis_index("c")
    sid = lax.axis_index("s")
    wid = cid * ns + sid

    @pl.when(wid < _BATCH)
    def _():
        pltpu.sync_copy(iw_hbm.at[wid], iw_v)
        row = iw_v[pl.ds(_SEQ - _LANES, _LANES)][_LANES - 1]
        base = pl.multiple_of((row // _TILE) * _TILE, _TILE)
        off = row - base
        blk_cp = pltpu.async_copy(
            tt_hbm.at[pl.ds(0, _HALF), pl.ds(base, _TILE)],
            blk_v.at[pl.ds(0, _HALF)], sem)
        blk_cp2 = pltpu.async_copy(
            tt_hbm.at[pl.ds(_HALF, _EMB - _HALF), pl.ds(base, _TILE)],
            blk_v.at[pl.ds(_HALF, _EMB - _HALF)], sem2)
        pltpu.async_copy(w_hbm.at[0], w_v, wsem).wait()
        pltpu.sync_copy(b_hbm, b_v.at[pl.ds(0, 1)])
        blk_cp.wait()
        blk_cp2.wait()
        col = jnp.full((_LANES,), off, jnp.int32)
        lanes = lax.iota(jnp.int32, _LANES)
        acc = jnp.zeros((_LANES,), jnp.float32)
        for j in range(_NCHUNK):
            vals = plsc.load_gather(blk_v, [lanes + (j * _LANES), col])
            acc = acc + vals * w_v[pl.ds(j * _LANES, _LANES)]
        tail = plsc.load_gather(blk_v, [lanes + _TAIL_OFF, col])
        tail_w = jnp.where(lanes >= _NCHUNK * _LANES - _TAIL_OFF,
                           w_v[pl.ds(_TAIL_OFF, _LANES)],
                           jnp.zeros((_LANES,), jnp.float32))
        acc = acc + tail * tail_w
        dnums = lax.GatherDimensionNumbers(
            offset_dims=(), collapsed_slice_dims=(0,), start_index_map=(0,))
        for sh in (8, 4, 2, 1):
            perm = lanes ^ sh
            acc = acc + lax.gather(
                acc, perm[:, None], dnums, slice_sizes=(1,),
                mode=lax.GatherScatterMode.PROMISE_IN_BOUNDS)
        bias = b_v[pl.ds(0, _LANES)][0]
        out_v[...] = 1.0 / (1.0 + jnp.exp(-(acc + bias)))
        pltpu.sync_copy(out_v, out_hbm.at[wid])


def kernel(input_words, table, fc_w, fc_b):
    mesh = plsc.VectorSubcoreMesh(core_axis_name="c", subcore_axis_name="s")
    sc_fn = functools.partial(
        pl.kernel,
        mesh=mesh,
        compiler_params=pltpu.CompilerParams(needs_layout_passes=False),
        out_type=jax.ShapeDtypeStruct((_BATCH, _LANES), jnp.float32),
        scratch_types=[
            pltpu.VMEM((_SEQ,), jnp.int32),
            pltpu.VMEM((_EMB, _TILE), jnp.float32),
            pltpu.VMEM((_EMB,), jnp.float32),
            pltpu.VMEM((_LANES,), jnp.float32),
            pltpu.VMEM((_LANES,), jnp.float32),
            pltpu.SemaphoreType.DMA,
            pltpu.SemaphoreType.DMA,
            pltpu.SemaphoreType.DMA,
        ],
    )(_sc_body)
    out2d = sc_fn(table.T, input_words.astype(jnp.int32),
                  fc_w.astype(jnp.float32), fc_b.astype(jnp.float32))
    return out2d[:, 0]


# speedup vs baseline: 1.0051x; 1.0045x over previous
import functools

import jax
import jax.numpy as jnp
from jax import lax
from jax.experimental import pallas as pl
from jax.experimental.pallas import tpu as pltpu
from jax.experimental.pallas import tpu_sc as plsc

_EMB = 300
_LANES = 16
_TILE = 128
_NCHUNK = _EMB // _LANES
_TAIL_OFF = _EMB - _LANES
_BATCH = 25
_SEQ = 200
_IW_OFF = 128
_IW_LEN = _SEQ - _IW_OFF


def _sc_body(tt_hbm, iw_hbm, w_hbm, b_hbm, out_hbm,
             iw_v, blk_v, w_v, b_v, out_v, sem, wsem):
    ns = plsc.get_sparse_core_info().num_subcores
    cid = lax.axis_index("c")
    sid = lax.axis_index("s")
    wid = cid * ns + sid

    @pl.when(wid < _BATCH)
    def _():
        pltpu.sync_copy(iw_hbm.at[wid], iw_v)
        row = iw_v[pl.ds(_SEQ - _LANES, _LANES)][_LANES - 1]
        base = pl.multiple_of((row // _TILE) * _TILE, _TILE)
        off = row - base
        blk_cp = pltpu.async_copy(tt_hbm.at[:, pl.ds(base, _TILE)], blk_v, sem)
        pltpu.async_copy(w_hbm.at[0], w_v, wsem).wait()
        pltpu.sync_copy(b_hbm, b_v.at[pl.ds(0, 1)])
        blk_cp.wait()
        col = jnp.full((_LANES,), off, jnp.int32)
        lanes = lax.iota(jnp.int32, _LANES)
        acc = jnp.zeros((_LANES,), jnp.float32)
        for j in range(_NCHUNK):
            vals = plsc.load_gather(blk_v, [lanes + (j * _LANES), col])
            acc = acc + vals * w_v[pl.ds(j * _LANES, _LANES)]
        tail = plsc.load_gather(blk_v, [lanes + _TAIL_OFF, col])
        tail_w = jnp.where(lanes >= _NCHUNK * _LANES - _TAIL_OFF,
                           w_v[pl.ds(_TAIL_OFF, _LANES)],
                           jnp.zeros((_LANES,), jnp.float32))
        acc = acc + tail * tail_w
        dnums = lax.GatherDimensionNumbers(
            offset_dims=(), collapsed_slice_dims=(0,), start_index_map=(0,))
        for sh in (8, 4, 2, 1):
            perm = lanes ^ sh
            acc = acc + lax.gather(
                acc, perm[:, None], dnums, slice_sizes=(1,),
                mode=lax.GatherScatterMode.PROMISE_IN_BOUNDS)
        bias = b_v[pl.ds(0, _LANES)][0]
        out_v[...] = 1.0 / (1.0 + jnp.exp(-(acc + bias)))
        pltpu.sync_copy(out_v, out_hbm.at[wid])


def kernel(input_words, table, fc_w, fc_b):
    mesh = plsc.VectorSubcoreMesh(core_axis_name="c", subcore_axis_name="s")
    sc_fn = functools.partial(
        pl.kernel,
        mesh=mesh,
        compiler_params=pltpu.CompilerParams(needs_layout_passes=False),
        out_type=jax.ShapeDtypeStruct((_BATCH, _LANES), jnp.float32),
        scratch_types=[
            pltpu.VMEM((_SEQ,), jnp.int32),
            pltpu.VMEM((_EMB, _TILE), jnp.float32),
            pltpu.VMEM((_EMB,), jnp.float32),
            pltpu.VMEM((_LANES,), jnp.float32),
            pltpu.VMEM((_LANES,), jnp.float32),
            pltpu.SemaphoreType.DMA,
            pltpu.SemaphoreType.DMA,
        ],
    )(_sc_body)
    out2d = sc_fn(table.T, input_words.astype(jnp.int32),
                  fc_w.astype(jnp.float32), fc_b.astype(jnp.float32))
    return out2d[:, 0]
